# baseline (speedup 1.0000x reference)
import functools

import jax
import jax.numpy as jnp
from jax import lax
from jax.experimental import pallas as pl
from jax.experimental.pallas import tpu as pltpu
from jax.experimental.pallas import tpu_sc as plsc

NC = 2
NS = 16
NW = NC * NS

F32 = jnp.float32


def _stage_rows(src, dst, s, R):
    rpt8 = (R // NS) // 8 * 8
    tail = R - NS * rpt8
    pltpu.sync_copy(src.at[pl.ds(s * rpt8, rpt8)],
                    dst.at[pl.ds(s * rpt8, rpt8)])
    if tail:
        @pl.when(s == 0)
        def _():
            pltpu.sync_copy(src.at[pl.ds(NS * rpt8, tail)],
                            dst.at[pl.ds(NS * rpt8, tail)])


def _sc_gather(table, idx_flat, chunk, sub=80, stage=True):
    R, D = table.shape
    dt = table.dtype
    T = idx_flat.shape[0]
    assert T % (NW * chunk) == 0 and chunk % sub == 0
    per_w = T // NW
    nch = per_w // chunk
    nsub = chunk // sub
    assert nch % 2 == 0 and nch >= 4

    mesh = plsc.VectorSubcoreMesh(core_axis_name="c", subcore_axis_name="s")

    @functools.partial(
        pl.kernel,
        out_type=jax.ShapeDtypeStruct((T, D), dt),
        mesh=mesh,
        scratch_types=[
            pltpu.VMEM_SHARED((R, D), dt) if stage else None,
            [pltpu.VMEM((chunk,), jnp.int32) for _ in range(2)],
            [pltpu.VMEM((chunk, D), dt) for _ in range(2)],
            [pltpu.SemaphoreType.DMA for _ in range(6)],
        ],
    )
    def k(tab_hbm, idx_hbm, out_hbm, tab_sp, idx_v, rows_v, sems):
        semi = sems[0:2]
        semg = sems[2:4]
        semo = sems[4:6]
        c = lax.axis_index("c")
        s = lax.axis_index("s")
        if stage:
            _stage_rows(tab_hbm, tab_sp, s, R)
            plsc.subcore_barrier()
            src = tab_sp
        else:
            src = tab_hbm
        w = s * NC + c
        w0 = w * per_w

        def idx_load(j, b):
            pltpu.async_copy(idx_hbm.at[pl.ds(w0 + j * chunk, chunk)],
                             idx_v[b], semi[b])

        def gath_start(b):
            for t in range(nsub):
                pltpu.async_copy(
                    src.at[idx_v[b].at[pl.ds(t * sub, sub)]],
                    rows_v[b].at[pl.ds(t * sub, sub)], semg[b])

        def gath_wait(b):
            for t in range(nsub):
                pltpu.make_async_copy(
                    src.at[idx_v[b].at[pl.ds(t * sub, sub)]],
                    rows_v[b].at[pl.ds(t * sub, sub)], semg[b]).wait()

        def out(j, b):
            return pltpu.make_async_copy(rows_v[b],
                                         out_hbm.at[pl.ds(w0 + j * chunk,
                                                          chunk)], semo[b])

        idx_load(0, 0)

        def pair(i, carry):
            for b in (0, 1):
                j = 2 * i + b
                nb = 1 - b
                pltpu.make_async_copy(idx_hbm.at[pl.ds(w0, chunk)],
                                      idx_v[b], semi[b]).wait()

                @pl.when(j >= 2)
                def _():
                    out(j, b).wait()

                @pl.when(j >= 1)
                def _():
                    gath_wait(nb)
                    out(j - 1, nb).start()

                gath_start(b)

                @pl.when(j + 1 < nch)
                def _():
                    idx_load(j + 1, nb)

            return carry

        lax.fori_loop(0, nch // 2, pair, 0)
        gath_wait(1)
        out(nch - 1, 1).start()
        out(nch - 2, 0).wait()
        out(nch - 1, 1).wait()

    return k(table, idx_flat)


def _sc_scatter_add(m, dst, h, zeros, chunk):
    E, D = m.shape
    R = h.shape[0]
    assert E % (NW * chunk) == 0
    per_w = E // NW
    nch = per_w // chunk
    assert nch >= 3

    mesh = plsc.VectorSubcoreMesh(core_axis_name="c", subcore_axis_name="s")

    @functools.partial(
        pl.kernel,
        out_type=[jax.ShapeDtypeStruct((R, D), F32),
                  jax.ShapeDtypeStruct((R, D), F32)],
        mesh=mesh,
        scratch_types=[
            pltpu.VMEM_SHARED((R, D), F32),
            [pltpu.VMEM((chunk, D), F32) for _ in range(4)],
            [pltpu.VMEM((chunk,), jnp.int32) for _ in range(4)],
            [pltpu.SemaphoreType.DMA for _ in range(12)],
        ],
    )
    def k(m_hbm, dst_hbm, h_hbm, z_hbm, out0_hbm, out1_hbm,
          acc_sp, m_v, idx_v, sems):
        semm = sems[0:4]
        semi = sems[4:8]
        sems_ = sems[8:12]
        c = lax.axis_index("c")
        s = lax.axis_index("s")

        @pl.when(c == 0)
        def _():
            _stage_rows(h_hbm, acc_sp, s, R)

        @pl.when(c != 0)
        def _():
            _stage_rows(z_hbm, acc_sp, s, R)

        plsc.subcore_barrier()
        w = s * NC + c
        w0 = w * per_w

        def loads(j, b):
            pltpu.async_copy(m_hbm.at[pl.ds(w0 + j * chunk, chunk)],
                             m_v[b], semm[b])
            pltpu.async_copy(dst_hbm.at[pl.ds(w0 + j * chunk, chunk)],
                             idx_v[b], semi[b])

        def wait_loads(b):
            pltpu.make_async_copy(m_hbm.at[pl.ds(w0, chunk)], m_v[b],
                                  semm[b]).wait()
            pltpu.make_async_copy(dst_hbm.at[pl.ds(w0, chunk)], idx_v[b],
                                  semi[b]).wait()

        def scat(b):
            return pltpu.make_async_copy(m_v[b], acc_sp.at[idx_v[b]],
                                         sems_[b])

        loads(0, 0)
        loads(1, 1)

        def step(j, b):
            wait_loads(b)

            @pl.when(j >= 2)
            def _():
                scat((b + 2) % 4).wait()

            scat(b).start(add=True)

            @pl.when(j + 2 < nch)
            def _():
                loads(j + 2, (b + 2) % 4)

        def quad(i, carry):
            for b in (0, 1, 2, 3):
                step(4 * i + b, b)
            return carry

        lax.fori_loop(0, nch // 4, quad, 0)
        for j in range((nch // 4) * 4, nch):
            step(j, j % 4)
        scat((nch - 1) % 4).wait()
        scat((nch - 2) % 4).wait()
        plsc.subcore_barrier()

        @pl.when(c == 0)
        def _():
            _stage_rows(acc_sp, out0_hbm, s, R)

        @pl.when(c != 0)
        def _():
            _stage_rows(acc_sp, out1_hbm, s, R)

    return k(m, dst, h, zeros)


GCHUNK = 80
SCHUNK = 40


def _silu(x):
    return x * jax.nn.sigmoid(x)


def _tc_input_mlp(x, W, b, blk):
    N, D = x.shape
    H = W.shape[1]

    def body(x_r, w_r, b_r, o_r):
        z = jnp.dot(x_r[...], w_r[...], preferred_element_type=F32) + b_r[...]
        o_r[...] = _silu(z)

    return pl.pallas_call(
        body,
        grid=(N // blk,),
        in_specs=[
            pl.BlockSpec((blk, D), lambda i: (i, 0)),
            pl.BlockSpec((D, H), lambda i: (0, 0)),
            pl.BlockSpec((1, H), lambda i: (0, 0)),
        ],
        out_specs=pl.BlockSpec((blk, H), lambda i: (i, 0)),
        out_shape=jax.ShapeDtypeStruct((N, H), F32),
        compiler_params=pltpu.CompilerParams(
            dimension_semantics=("arbitrary",)),
    )(x, W, b.reshape(1, H))


def _tc_d2(pp, E, blk):
    nb = E // blk

    def body(ps_r, pd_r, o_r):
        d = ps_r[...].astype(F32) - pd_r[...].astype(F32)
        o_r[...] = jnp.sum(d * d, axis=1, keepdims=True)

    return pl.pallas_call(
        body,
        grid=(nb,),
        in_specs=[
            pl.BlockSpec((blk, 128), lambda i: (i, 0)),
            pl.BlockSpec((blk, 128), lambda i, nb=nb: (i + nb, 0)),
        ],
        out_specs=pl.BlockSpec((blk, 1), lambda i: (i, 0)),
        out_shape=jax.ShapeDtypeStruct((E, 1), F32),
        compiler_params=pltpu.CompilerParams(
            dimension_semantics=("arbitrary",)),
    )(pp, pp)


def _tc_edge_mlp(hsd, d2, A, B, c1, b1, W2, b2, sW, sb, E, H, blk, d2_off=0):
    nb = E // blk

    def body(hs_r, hd_r, d2_r, A_r, B_r, c1_r, b1_r, W2_r, b2_r, sW_r, sb_r,
             o_r):
        z = jnp.dot(hs_r[...].astype(BF), A_r[...], preferred_element_type=F32)
        z = z + jnp.dot(hd_r[...].astype(BF), B_r[...],
                        preferred_element_type=F32)
        z = z + d2_r[...] * c1_r[...] + b1_r[...]
        m1 = _silu(z)
        z2 = jnp.dot(m1.astype(jnp.bfloat16), W2_r[...],
                     preferred_element_type=F32) + b2_r[...]
        m = _silu(z2)
        g = jax.nn.sigmoid(jnp.dot(m, sW_r[...],
                                   preferred_element_type=F32) + sb_r[...])
        o_r[...] = m * g

    return pl.pallas_call(
        body,
        grid=(nb,),
        in_specs=[
            pl.BlockSpec((blk, H), lambda i: (i, 0)),
            pl.BlockSpec((blk, H), lambda i, nb=nb: (i + nb, 0)),
            pl.BlockSpec((blk, 1), lambda i, o=d2_off: (i + o, 0)),
            pl.BlockSpec((H, H), lambda i: (0, 0)),
            pl.BlockSpec((H, H), lambda i: (0, 0)),
            pl.BlockSpec((1, H), lambda i: (0, 0)),
            pl.BlockSpec((1, H), lambda i: (0, 0)),
            pl.BlockSpec((H, H), lambda i: (0, 0)),
            pl.BlockSpec((1, H), lambda i: (0, 0)),
            pl.BlockSpec((H, 1), lambda i: (0, 0)),
            pl.BlockSpec((1, 1), lambda i: (0, 0)),
        ],
        out_specs=pl.BlockSpec((blk, H), lambda i: (i, 0)),
        out_shape=jax.ShapeDtypeStruct((E, H), F32),
        compiler_params=pltpu.CompilerParams(
            dimension_semantics=("arbitrary",)),
    )(hsd, hsd, d2, A, B, c1, b1, W2, b2, sW, sb)


def _tc_node_mlp(p0, p1, h, W1, b1, W2, b2, blk):
    N, H = h.shape

    def body(p0_r, p1_r, h_r, W1_r, b1_r, W2_r, b2_r, o_r):
        hi = p0_r[...] + p1_r[...]
        a = _silu(jnp.dot(hi, W1_r[...], preferred_element_type=F32)
                  + b1_r[...])
        o_r[...] = (jnp.dot(a, W2_r[...], preferred_element_type=F32)
                    + b2_r[...] + h_r[...])

    return pl.pallas_call(
        body,
        grid=(N // blk,),
        in_specs=[
            pl.BlockSpec((blk, H), lambda i: (i, 0)),
            pl.BlockSpec((blk, H), lambda i: (i, 0)),
            pl.BlockSpec((blk, H), lambda i: (i, 0)),
            pl.BlockSpec((H, H), lambda i: (0, 0)),
            pl.BlockSpec((1, H), lambda i: (0, 0)),
            pl.BlockSpec((H, H), lambda i: (0, 0)),
            pl.BlockSpec((1, H), lambda i: (0, 0)),
        ],
        out_specs=pl.BlockSpec((blk, H), lambda i: (i, 0)),
        out_shape=jax.ShapeDtypeStruct((N, H), F32),
        compiler_params=pltpu.CompilerParams(
            dimension_semantics=("arbitrary",)),
    )(p0, p1, h, W1, b1.reshape(1, H), W2, b2.reshape(1, H))


def _tc_readout(h, oW1, ob1, oW2, ob2, rW1, rb1, rW2, rb2, blk):
    N, H = h.shape
    T = rW2.shape[1]
    nb = N // blk

    def body(h_r, oW1_r, ob1_r, oW2_r, ob2_r, rW1_r, rb1_r, rW2_r, rb2_r,
             o_r, sum_acc, max_acc):
        i = pl.program_id(0)
        a = _silu(jnp.dot(h_r[...], oW1_r[...], preferred_element_type=F32)
                  + ob1_r[...])
        ho = jnp.dot(a, oW2_r[...], preferred_element_type=F32) + ob2_r[...]
        ps = jnp.sum(ho, axis=0, keepdims=True)
        pm = jnp.max(ho, axis=0, keepdims=True)

        @pl.when(i == 0)
        def _():
            sum_acc[...] = ps
            max_acc[...] = pm

        @pl.when(i > 0)
        def _():
            sum_acc[...] = sum_acc[...] + ps
            max_acc[...] = jnp.maximum(max_acc[...], pm)

        @pl.when(i == nb - 1)
        def _():
            s = sum_acc[...]
            mx = max_acc[...]
            r = jnp.concatenate([s, s * (1.0 / N), mx], axis=1)
            t = jnp.dot(r, rW1_r[...], preferred_element_type=F32) + rb1_r[...]
            t = jnp.maximum(t, 0.0)
            o_r[...] = (jnp.dot(t, rW2_r[...], preferred_element_type=F32)
                        + rb2_r[...])

    return pl.pallas_call(
        body,
        grid=(nb,),
        in_specs=[
            pl.BlockSpec((blk, H), lambda i: (i, 0)),
            pl.BlockSpec((H, H), lambda i: (0, 0)),
            pl.BlockSpec((1, H), lambda i: (0, 0)),
            pl.BlockSpec((H, H), lambda i: (0, 0)),
            pl.BlockSpec((1, H), lambda i: (0, 0)),
            pl.BlockSpec((3 * H, H), lambda i: (0, 0)),
            pl.BlockSpec((1, H), lambda i: (0, 0)),
            pl.BlockSpec((H, T), lambda i: (0, 0)),
            pl.BlockSpec((1, T), lambda i: (0, 0)),
        ],
        out_specs=pl.BlockSpec((1, T), lambda i: (0, 0)),
        out_shape=jax.ShapeDtypeStruct((1, T), F32),
        scratch_shapes=[pltpu.VMEM((1, H), F32), pltpu.VMEM((1, H), F32)],
        compiler_params=pltpu.CompilerParams(
            dimension_semantics=("arbitrary",)),
    )(h, oW1, ob1.reshape(1, H), oW2, ob2.reshape(1, H),
      rW1, rb1.reshape(1, H), rW2, rb2.reshape(1, T))


BF = jnp.bfloat16
EBLK = 2000
NBLK = 2000


def kernel(x, pos, edge_index, params):
    N, D = x.shape
    E = edge_index.shape[1]
    H = params["in_W"].shape[1]

    h = _tc_input_mlp(x, params["in_W"], params["in_b"], NBLK)

    idx_flat = edge_index.reshape(2 * E)
    dst = edge_index[1]
    pos_pad = jnp.concatenate(
        [pos, jnp.zeros((N, 128 - pos.shape[1]), F32)], axis=1)

    pp = _sc_gather(pos_pad, idx_flat, 400, stage=False)
    d2 = _tc_d2(pp, E, EBLK)

    zeros = jnp.zeros((N, H), F32)
    EA = E * 3 // 5
    assert EA % (NW * GCHUNK) == 0 and (E - EA) % (NW * GCHUNK) == 0
    idxA = jnp.concatenate([edge_index[0, :EA], edge_index[1, :EA]])
    idxB = jnp.concatenate([edge_index[0, EA:], edge_index[1, EA:]])
    dstA = edge_index[1, :EA]
    dstB = edge_index[1, EA:]
    for lp in params["layers"]:
        eargs = (lp["mW1"][:H].astype(BF), lp["mW1"][H:2 * H].astype(BF),
                 lp["mW1"][2 * H:2 * H + 1],
                 lp["mb1"].reshape(1, H), lp["mW2"].astype(BF),
                 lp["mb2"].reshape(1, H),
                 lp["sW"], lp["sb"].reshape(1, 1))
        hsdA = _sc_gather(h, idxA, 400, stage=False)
        hsdB = _sc_gather(h, idxB, 400, stage=False)
        mA = _tc_edge_mlp(hsdA, d2, *eargs, EA, H, EBLK)
        mB = _tc_edge_mlp(hsdB, d2, *eargs, E - EA, H, EBLK,
                          d2_off=EA // EBLK)
        q0, q1 = _sc_scatter_add(mA, dstA, h, zeros, SCHUNK)
        p0, p1 = _sc_scatter_add(mB, dstB, q0, q1, SCHUNK)
        h = _tc_node_mlp(p0, p1, h, lp["uW1"], lp["ub1"], lp["uW2"],
                         lp["ub2"], NBLK)

    return _tc_readout(h, params["oW1"], params["ob1"], params["oW2"],
                       params["ob2"], params["rW1"], params["rb1"],
                       params["rW2"], params["rb2"], NBLK)


# speedup vs baseline: 1.1879x; 1.1879x over previous
import functools

import jax
import jax.numpy as jnp
from jax import lax
from jax.experimental import pallas as pl
from jax.experimental.pallas import tpu as pltpu
from jax.experimental.pallas import tpu_sc as plsc

NC = 2
NS = 16
NW = NC * NS

F32 = jnp.float32


def _stage_rows(src, dst, s, R):
    rpt8 = (R // NS) // 8 * 8
    tail = R - NS * rpt8
    pltpu.sync_copy(src.at[pl.ds(s * rpt8, rpt8)],
                    dst.at[pl.ds(s * rpt8, rpt8)])
    if tail:
        @pl.when(s == 0)
        def _():
            pltpu.sync_copy(src.at[pl.ds(NS * rpt8, tail)],
                            dst.at[pl.ds(NS * rpt8, tail)])


def _sc_gather(table, idx_flat, chunk, sub=80, stage=True):
    R, D = table.shape
    dt = table.dtype
    T = idx_flat.shape[0]
    assert T % (NW * chunk) == 0 and chunk % sub == 0
    per_w = T // NW
    nch = per_w // chunk
    nsub = chunk // sub
    assert nch % 2 == 0 and nch >= 4

    mesh = plsc.VectorSubcoreMesh(core_axis_name="c", subcore_axis_name="s")

    @functools.partial(
        pl.kernel,
        out_type=jax.ShapeDtypeStruct((T, D), dt),
        mesh=mesh,
        scratch_types=[
            pltpu.VMEM_SHARED((R, D), dt) if stage else None,
            [pltpu.VMEM((chunk,), jnp.int32) for _ in range(2)],
            [pltpu.VMEM((chunk, D), dt) for _ in range(2)],
            [pltpu.SemaphoreType.DMA for _ in range(6)],
        ],
    )
    def k(tab_hbm, idx_hbm, out_hbm, tab_sp, idx_v, rows_v, sems):
        semi = sems[0:2]
        semg = sems[2:4]
        semo = sems[4:6]
        c = lax.axis_index("c")
        s = lax.axis_index("s")
        if stage:
            _stage_rows(tab_hbm, tab_sp, s, R)
            plsc.subcore_barrier()
            src = tab_sp
        else:
            src = tab_hbm
        w = s * NC + c
        w0 = w * per_w

        def idx_load(j, b):
            pltpu.async_copy(idx_hbm.at[pl.ds(w0 + j * chunk, chunk)],
                             idx_v[b], semi[b])

        def gath_start(b):
            for t in range(nsub):
                pltpu.async_copy(
                    src.at[idx_v[b].at[pl.ds(t * sub, sub)]],
                    rows_v[b].at[pl.ds(t * sub, sub)], semg[b])

        def gath_wait(b):
            for t in range(nsub):
                pltpu.make_async_copy(
                    src.at[idx_v[b].at[pl.ds(t * sub, sub)]],
                    rows_v[b].at[pl.ds(t * sub, sub)], semg[b]).wait()

        def out(j, b):
            return pltpu.make_async_copy(rows_v[b],
                                         out_hbm.at[pl.ds(w0 + j * chunk,
                                                          chunk)], semo[b])

        idx_load(0, 0)

        def pair(i, carry):
            for b in (0, 1):
                j = 2 * i + b
                nb = 1 - b
                pltpu.make_async_copy(idx_hbm.at[pl.ds(w0, chunk)],
                                      idx_v[b], semi[b]).wait()

                @pl.when(j >= 2)
                def _():
                    out(j, b).wait()

                @pl.when(j >= 1)
                def _():
                    gath_wait(nb)
                    out(j - 1, nb).start()

                gath_start(b)

                @pl.when(j + 1 < nch)
                def _():
                    idx_load(j + 1, nb)

            return carry

        lax.fori_loop(0, nch // 2, pair, 0)
        gath_wait(1)
        out(nch - 1, 1).start()
        out(nch - 2, 0).wait()
        out(nch - 1, 1).wait()

    return k(table, idx_flat)


def _sc_scatter_add(m, dst, h, zeros, chunk):
    E, D = m.shape
    R = h.shape[0]
    assert E % (NW * chunk) == 0
    per_w = E // NW
    nch = per_w // chunk
    assert nch >= 3

    mesh = plsc.VectorSubcoreMesh(core_axis_name="c", subcore_axis_name="s")

    @functools.partial(
        pl.kernel,
        out_type=[jax.ShapeDtypeStruct((R, D), F32),
                  jax.ShapeDtypeStruct((R, D), F32)],
        mesh=mesh,
        scratch_types=[
            pltpu.VMEM_SHARED((R, D), F32),
            [pltpu.VMEM((chunk, D), F32) for _ in range(4)],
            [pltpu.VMEM((chunk,), jnp.int32) for _ in range(4)],
            [pltpu.SemaphoreType.DMA for _ in range(12)],
        ],
    )
    def k(m_hbm, dst_hbm, h_hbm, z_hbm, out0_hbm, out1_hbm,
          acc_sp, m_v, idx_v, sems):
        semm = sems[0:4]
        semi = sems[4:8]
        sems_ = sems[8:12]
        c = lax.axis_index("c")
        s = lax.axis_index("s")

        @pl.when(c == 0)
        def _():
            _stage_rows(h_hbm, acc_sp, s, R)

        @pl.when(c != 0)
        def _():
            _stage_rows(z_hbm, acc_sp, s, R)

        plsc.subcore_barrier()
        w = s * NC + c
        w0 = w * per_w

        def loads(j, b):
            pltpu.async_copy(m_hbm.at[pl.ds(w0 + j * chunk, chunk)],
                             m_v[b], semm[b])
            pltpu.async_copy(dst_hbm.at[pl.ds(w0 + j * chunk, chunk)],
                             idx_v[b], semi[b])

        def wait_loads(b):
            pltpu.make_async_copy(m_hbm.at[pl.ds(w0, chunk)], m_v[b],
                                  semm[b]).wait()
            pltpu.make_async_copy(dst_hbm.at[pl.ds(w0, chunk)], idx_v[b],
                                  semi[b]).wait()

        def scat(b):
            return pltpu.make_async_copy(m_v[b], acc_sp.at[idx_v[b]],
                                         sems_[b])

        loads(0, 0)
        loads(1, 1)

        def step(j, b):
            wait_loads(b)

            @pl.when(j >= 2)
            def _():
                scat((b + 2) % 4).wait()

            scat(b).start(add=True)

            @pl.when(j + 2 < nch)
            def _():
                loads(j + 2, (b + 2) % 4)

        def quad(i, carry):
            for b in (0, 1, 2, 3):
                step(4 * i + b, b)
            return carry

        lax.fori_loop(0, nch // 4, quad, 0)
        for j in range((nch // 4) * 4, nch):
            step(j, j % 4)
        scat((nch - 1) % 4).wait()
        scat((nch - 2) % 4).wait()
        plsc.subcore_barrier()

        @pl.when(c == 0)
        def _():
            _stage_rows(acc_sp, out0_hbm, s, R)

        @pl.when(c != 0)
        def _():
            _stage_rows(acc_sp, out1_hbm, s, R)

    return k(m, dst, h, zeros)


GCHUNK = 80
SCHUNK = 40


def _silu(x):
    return x * jax.nn.sigmoid(x)


def _tc_input_mlp(x, W, b, blk):
    N, D = x.shape
    H = W.shape[1]

    def body(x_r, w_r, b_r, o_r):
        z = jnp.dot(x_r[...], w_r[...], preferred_element_type=F32) + b_r[...]
        o_r[...] = _silu(z)

    return pl.pallas_call(
        body,
        grid=(N // blk,),
        in_specs=[
            pl.BlockSpec((blk, D), lambda i: (i, 0)),
            pl.BlockSpec((D, H), lambda i: (0, 0)),
            pl.BlockSpec((1, H), lambda i: (0, 0)),
        ],
        out_specs=pl.BlockSpec((blk, H), lambda i: (i, 0)),
        out_shape=jax.ShapeDtypeStruct((N, H), F32),
        compiler_params=pltpu.CompilerParams(
            dimension_semantics=("arbitrary",)),
    )(x, W, b.reshape(1, H))


def _tc_d2(pp, E, blk):
    nb = E // blk

    def body(ps_r, pd_r, o_r):
        d = ps_r[...].astype(F32) - pd_r[...].astype(F32)
        o_r[...] = jnp.sum(d * d, axis=1, keepdims=True)

    return pl.pallas_call(
        body,
        grid=(nb,),
        in_specs=[
            pl.BlockSpec((blk, 128), lambda i: (i, 0)),
            pl.BlockSpec((blk, 128), lambda i, nb=nb: (i + nb, 0)),
        ],
        out_specs=pl.BlockSpec((blk, 1), lambda i: (i, 0)),
        out_shape=jax.ShapeDtypeStruct((E, 1), F32),
        compiler_params=pltpu.CompilerParams(
            dimension_semantics=("arbitrary",)),
    )(pp, pp)


def _tc_edge_mlp(hsd, d2, A, B, c1, b1, W2, b2, sW, sb, E, H, blk, d2_off=0):
    nb = E // blk

    def body(hs_r, hd_r, d2_r, A_r, B_r, c1_r, b1_r, W2_r, b2_r, sW_r, sb_r,
             o_r):
        z = jnp.dot(hs_r[...].astype(BF), A_r[...], preferred_element_type=F32)
        z = z + jnp.dot(hd_r[...].astype(BF), B_r[...],
                        preferred_element_type=F32)
        z = z + d2_r[...] * c1_r[...] + b1_r[...]
        m1 = _silu(z)
        z2 = jnp.dot(m1.astype(jnp.bfloat16), W2_r[...],
                     preferred_element_type=F32) + b2_r[...]
        m = _silu(z2)
        g = jax.nn.sigmoid(jnp.dot(m, sW_r[...],
                                   preferred_element_type=F32) + sb_r[...])
        o_r[...] = m * g

    return pl.pallas_call(
        body,
        grid=(nb,),
        in_specs=[
            pl.BlockSpec((blk, H), lambda i: (i, 0)),
            pl.BlockSpec((blk, H), lambda i, nb=nb: (i + nb, 0)),
            pl.BlockSpec((blk, 1), lambda i, o=d2_off: (i + o, 0)),
            pl.BlockSpec((H, H), lambda i: (0, 0)),
            pl.BlockSpec((H, H), lambda i: (0, 0)),
            pl.BlockSpec((1, H), lambda i: (0, 0)),
            pl.BlockSpec((1, H), lambda i: (0, 0)),
            pl.BlockSpec((H, H), lambda i: (0, 0)),
            pl.BlockSpec((1, H), lambda i: (0, 0)),
            pl.BlockSpec((H, 1), lambda i: (0, 0)),
            pl.BlockSpec((1, 1), lambda i: (0, 0)),
        ],
        out_specs=pl.BlockSpec((blk, H), lambda i: (i, 0)),
        out_shape=jax.ShapeDtypeStruct((E, H), F32),
        compiler_params=pltpu.CompilerParams(
            dimension_semantics=("arbitrary",)),
    )(hsd, hsd, d2, A, B, c1, b1, W2, b2, sW, sb)


def _tc_node_mlp(p0, p1, h, W1, b1, W2, b2, blk):
    N, H = h.shape

    def body(p0_r, p1_r, h_r, W1_r, b1_r, W2_r, b2_r, o_r):
        hi = p0_r[...] + p1_r[...]
        a = _silu(jnp.dot(hi, W1_r[...], preferred_element_type=F32)
                  + b1_r[...])
        o_r[...] = (jnp.dot(a, W2_r[...], preferred_element_type=F32)
                    + b2_r[...] + h_r[...])

    return pl.pallas_call(
        body,
        grid=(N // blk,),
        in_specs=[
            pl.BlockSpec((blk, H), lambda i: (i, 0)),
            pl.BlockSpec((blk, H), lambda i: (i, 0)),
            pl.BlockSpec((blk, H), lambda i: (i, 0)),
            pl.BlockSpec((H, H), lambda i: (0, 0)),
            pl.BlockSpec((1, H), lambda i: (0, 0)),
            pl.BlockSpec((H, H), lambda i: (0, 0)),
            pl.BlockSpec((1, H), lambda i: (0, 0)),
        ],
        out_specs=pl.BlockSpec((blk, H), lambda i: (i, 0)),
        out_shape=jax.ShapeDtypeStruct((N, H), F32),
        compiler_params=pltpu.CompilerParams(
            dimension_semantics=("arbitrary",)),
    )(p0, p1, h, W1, b1.reshape(1, H), W2, b2.reshape(1, H))


def _tc_readout(h, oW1, ob1, oW2, ob2, rW1, rb1, rW2, rb2, blk):
    N, H = h.shape
    T = rW2.shape[1]
    nb = N // blk

    def body(h_r, oW1_r, ob1_r, oW2_r, ob2_r, rW1_r, rb1_r, rW2_r, rb2_r,
             o_r, sum_acc, max_acc):
        i = pl.program_id(0)
        a = _silu(jnp.dot(h_r[...], oW1_r[...], preferred_element_type=F32)
                  + ob1_r[...])
        ho = jnp.dot(a, oW2_r[...], preferred_element_type=F32) + ob2_r[...]
        ps = jnp.sum(ho, axis=0, keepdims=True)
        pm = jnp.max(ho, axis=0, keepdims=True)

        @pl.when(i == 0)
        def _():
            sum_acc[...] = ps
            max_acc[...] = pm

        @pl.when(i > 0)
        def _():
            sum_acc[...] = sum_acc[...] + ps
            max_acc[...] = jnp.maximum(max_acc[...], pm)

        @pl.when(i == nb - 1)
        def _():
            s = sum_acc[...]
            mx = max_acc[...]
            r = jnp.concatenate([s, s * (1.0 / N), mx], axis=1)
            t = jnp.dot(r, rW1_r[...], preferred_element_type=F32) + rb1_r[...]
            t = jnp.maximum(t, 0.0)
            o_r[...] = (jnp.dot(t, rW2_r[...], preferred_element_type=F32)
                        + rb2_r[...])

    return pl.pallas_call(
        body,
        grid=(nb,),
        in_specs=[
            pl.BlockSpec((blk, H), lambda i: (i, 0)),
            pl.BlockSpec((H, H), lambda i: (0, 0)),
            pl.BlockSpec((1, H), lambda i: (0, 0)),
            pl.BlockSpec((H, H), lambda i: (0, 0)),
            pl.BlockSpec((1, H), lambda i: (0, 0)),
            pl.BlockSpec((3 * H, H), lambda i: (0, 0)),
            pl.BlockSpec((1, H), lambda i: (0, 0)),
            pl.BlockSpec((H, T), lambda i: (0, 0)),
            pl.BlockSpec((1, T), lambda i: (0, 0)),
        ],
        out_specs=pl.BlockSpec((1, T), lambda i: (0, 0)),
        out_shape=jax.ShapeDtypeStruct((1, T), F32),
        scratch_shapes=[pltpu.VMEM((1, H), F32), pltpu.VMEM((1, H), F32)],
        compiler_params=pltpu.CompilerParams(
            dimension_semantics=("arbitrary",)),
    )(h, oW1, ob1.reshape(1, H), oW2, ob2.reshape(1, H),
      rW1, rb1.reshape(1, H), rW2, rb2.reshape(1, T))


BF = jnp.bfloat16
EBLK = 2000
NBLK = 2000


def kernel(x, pos, edge_index, params):
    N, D = x.shape
    E = edge_index.shape[1]
    H = params["in_W"].shape[1]

    h = _tc_input_mlp(x, params["in_W"], params["in_b"], NBLK)

    idx_flat = edge_index.reshape(2 * E)
    dst = edge_index[1]
    pos_pad = jnp.concatenate(
        [pos, jnp.zeros((N, 128 - pos.shape[1]), F32)], axis=1)

    pp = _sc_gather(pos_pad, idx_flat, GCHUNK)
    d2 = _tc_d2(pp, E, EBLK)

    zeros = jnp.zeros((N, H), F32)
    EA = E * 3 // 5
    assert EA % (NW * GCHUNK) == 0 and (E - EA) % (NW * GCHUNK) == 0
    idxA = jnp.concatenate([edge_index[0, :EA], edge_index[1, :EA]])
    idxB = jnp.concatenate([edge_index[0, EA:], edge_index[1, EA:]])
    dstA = edge_index[1, :EA]
    dstB = edge_index[1, EA:]
    for lp in params["layers"]:
        eargs = (lp["mW1"][:H].astype(BF), lp["mW1"][H:2 * H].astype(BF),
                 lp["mW1"][2 * H:2 * H + 1],
                 lp["mb1"].reshape(1, H), lp["mW2"].astype(BF),
                 lp["mb2"].reshape(1, H),
                 lp["sW"], lp["sb"].reshape(1, 1))
        hsdA = _sc_gather(h, idxA, GCHUNK)
        hsdB = _sc_gather(h, idxB, GCHUNK)
        mA = _tc_edge_mlp(hsdA, d2, *eargs, EA, H, EBLK)
        mB = _tc_edge_mlp(hsdB, d2, *eargs, E - EA, H, EBLK,
                          d2_off=EA // EBLK)
        q0, q1 = _sc_scatter_add(mA, dstA, h, zeros, SCHUNK)
        p0, p1 = _sc_scatter_add(mB, dstB, q0, q1, SCHUNK)
        h = _tc_node_mlp(p0, p1, h, lp["uW1"], lp["ub1"], lp["uW2"],
                         lp["ub2"], NBLK)

    return _tc_readout(h, params["oW1"], params["ob1"], params["oW2"],
                       params["ob2"], params["rW1"], params["rb1"],
                       params["rW2"], params["rb2"], NBLK)


# speedup vs baseline: 1.1952x; 1.0062x over previous
import functools

import jax
import jax.numpy as jnp
from jax import lax
from jax.experimental import pallas as pl
from jax.experimental.pallas import tpu as pltpu
from jax.experimental.pallas import tpu_sc as plsc

NC = 2
NS = 16
NW = NC * NS

F32 = jnp.float32


def _stage_rows(src, dst, s, R):
    rpt8 = (R // NS) // 8 * 8
    tail = R - NS * rpt8
    pltpu.sync_copy(src.at[pl.ds(s * rpt8, rpt8)],
                    dst.at[pl.ds(s * rpt8, rpt8)])
    if tail:
        @pl.when(s == 0)
        def _():
            pltpu.sync_copy(src.at[pl.ds(NS * rpt8, tail)],
                            dst.at[pl.ds(NS * rpt8, tail)])


def _sc_gather(table, idx_flat, chunk, sub=80, stage=True):
    R, D = table.shape
    dt = table.dtype
    T = idx_flat.shape[0]
    assert T % (NW * chunk) == 0 and chunk % sub == 0
    per_w = T // NW
    nch = per_w // chunk
    nsub = chunk // sub
    assert nch % 2 == 0 and nch >= 4

    mesh = plsc.VectorSubcoreMesh(core_axis_name="c", subcore_axis_name="s")

    @functools.partial(
        pl.kernel,
        out_type=jax.ShapeDtypeStruct((T, D), dt),
        mesh=mesh,
        scratch_types=[
            pltpu.VMEM_SHARED((R, D), dt) if stage else None,
            [pltpu.VMEM((chunk,), jnp.int32) for _ in range(2)],
            [pltpu.VMEM((chunk, D), dt) for _ in range(2)],
            [pltpu.SemaphoreType.DMA for _ in range(6)],
        ],
    )
    def k(tab_hbm, idx_hbm, out_hbm, tab_sp, idx_v, rows_v, sems):
        semi = sems[0:2]
        semg = sems[2:4]
        semo = sems[4:6]
        c = lax.axis_index("c")
        s = lax.axis_index("s")
        if stage:
            _stage_rows(tab_hbm, tab_sp, s, R)
            plsc.subcore_barrier()
            src = tab_sp
        else:
            src = tab_hbm
        w = s * NC + c
        w0 = w * per_w

        def idx_load(j, b):
            pltpu.async_copy(idx_hbm.at[pl.ds(w0 + j * chunk, chunk)],
                             idx_v[b], semi[b])

        def gath_start(b):
            for t in range(nsub):
                pltpu.async_copy(
                    src.at[idx_v[b].at[pl.ds(t * sub, sub)]],
                    rows_v[b].at[pl.ds(t * sub, sub)], semg[b])

        def gath_wait(b):
            for t in range(nsub):
                pltpu.make_async_copy(
                    src.at[idx_v[b].at[pl.ds(t * sub, sub)]],
                    rows_v[b].at[pl.ds(t * sub, sub)], semg[b]).wait()

        def out(j, b):
            return pltpu.make_async_copy(rows_v[b],
                                         out_hbm.at[pl.ds(w0 + j * chunk,
                                                          chunk)], semo[b])

        idx_load(0, 0)

        def pair(i, carry):
            for b in (0, 1):
                j = 2 * i + b
                nb = 1 - b
                pltpu.make_async_copy(idx_hbm.at[pl.ds(w0, chunk)],
                                      idx_v[b], semi[b]).wait()

                @pl.when(j >= 2)
                def _():
                    out(j, b).wait()

                @pl.when(j >= 1)
                def _():
                    gath_wait(nb)
                    out(j - 1, nb).start()

                gath_start(b)

                @pl.when(j + 1 < nch)
                def _():
                    idx_load(j + 1, nb)

            return carry

        lax.fori_loop(0, nch // 2, pair, 0)
        gath_wait(1)
        out(nch - 1, 1).start()
        out(nch - 2, 0).wait()
        out(nch - 1, 1).wait()

    return k(table, idx_flat)


def _sc_scatter_add(m, dst, h, zeros, chunk):
    E, D = m.shape
    R = h.shape[0]
    assert E % (NW * chunk) == 0
    per_w = E // NW
    nch = per_w // chunk
    assert nch >= 3

    mesh = plsc.VectorSubcoreMesh(core_axis_name="c", subcore_axis_name="s")

    @functools.partial(
        pl.kernel,
        out_type=[jax.ShapeDtypeStruct((R, D), F32),
                  jax.ShapeDtypeStruct((R, D), F32)],
        mesh=mesh,
        scratch_types=[
            pltpu.VMEM_SHARED((R, D), F32),
            [pltpu.VMEM((chunk, D), F32) for _ in range(6)],
            [pltpu.VMEM((chunk,), jnp.int32) for _ in range(6)],
            [pltpu.SemaphoreType.DMA for _ in range(18)],
        ],
    )
    def k(m_hbm, dst_hbm, h_hbm, z_hbm, out0_hbm, out1_hbm,
          acc_sp, m_v, idx_v, sems):
        semm = sems[0:6]
        semi = sems[6:12]
        sems_ = sems[12:18]
        c = lax.axis_index("c")
        s = lax.axis_index("s")

        @pl.when(c == 0)
        def _():
            _stage_rows(h_hbm, acc_sp, s, R)

        @pl.when(c != 0)
        def _():
            _stage_rows(z_hbm, acc_sp, s, R)

        plsc.subcore_barrier()
        w = s * NC + c
        w0 = w * per_w

        def loads(j, b):
            pltpu.async_copy(m_hbm.at[pl.ds(w0 + j * chunk, chunk)],
                             m_v[b], semm[b])
            pltpu.async_copy(dst_hbm.at[pl.ds(w0 + j * chunk, chunk)],
                             idx_v[b], semi[b])

        def wait_loads(b):
            pltpu.make_async_copy(m_hbm.at[pl.ds(w0, chunk)], m_v[b],
                                  semm[b]).wait()
            pltpu.make_async_copy(dst_hbm.at[pl.ds(w0, chunk)], idx_v[b],
                                  semi[b]).wait()

        def scat(b):
            return pltpu.make_async_copy(m_v[b], acc_sp.at[idx_v[b]],
                                         sems_[b])

        loads(0, 0)
        loads(1, 1)
        loads(2, 2)

        def step(j, b):
            wait_loads(b)

            @pl.when(j >= 3)
            def _():
                scat((b + 3) % 6).wait()

            scat(b).start(add=True)

            @pl.when(j + 3 < nch)
            def _():
                loads(j + 3, (b + 3) % 6)

        def six(i, carry):
            for b in (0, 1, 2, 3, 4, 5):
                step(6 * i + b, b)
            return carry

        lax.fori_loop(0, nch // 6, six, 0)
        for j in range((nch // 6) * 6, nch):
            step(j, j % 6)
        scat((nch - 1) % 6).wait()
        scat((nch - 2) % 6).wait()
        scat((nch - 3) % 6).wait()
        plsc.subcore_barrier()

        @pl.when(c == 0)
        def _():
            _stage_rows(acc_sp, out0_hbm, s, R)

        @pl.when(c != 0)
        def _():
            _stage_rows(acc_sp, out1_hbm, s, R)

    return k(m, dst, h, zeros)


GCHUNK = 80
SCHUNK = 40


def _silu(x):
    return x * jax.nn.sigmoid(x)


def _tc_input_mlp(x, W, b, blk):
    N, D = x.shape
    H = W.shape[1]

    def body(x_r, w_r, b_r, o_r):
        z = jnp.dot(x_r[...], w_r[...], preferred_element_type=F32) + b_r[...]
        o_r[...] = _silu(z)

    return pl.pallas_call(
        body,
        grid=(N // blk,),
        in_specs=[
            pl.BlockSpec((blk, D), lambda i: (i, 0)),
            pl.BlockSpec((D, H), lambda i: (0, 0)),
            pl.BlockSpec((1, H), lambda i: (0, 0)),
        ],
        out_specs=pl.BlockSpec((blk, H), lambda i: (i, 0)),
        out_shape=jax.ShapeDtypeStruct((N, H), F32),
        compiler_params=pltpu.CompilerParams(
            dimension_semantics=("arbitrary",)),
    )(x, W, b.reshape(1, H))


def _tc_d2(pp, E, blk):
    nb = E // blk

    def body(ps_r, pd_r, o_r):
        d = ps_r[...].astype(F32) - pd_r[...].astype(F32)
        o_r[...] = jnp.sum(d * d, axis=1, keepdims=True)

    return pl.pallas_call(
        body,
        grid=(nb,),
        in_specs=[
            pl.BlockSpec((blk, 128), lambda i: (i, 0)),
            pl.BlockSpec((blk, 128), lambda i, nb=nb: (i + nb, 0)),
        ],
        out_specs=pl.BlockSpec((blk, 1), lambda i: (i, 0)),
        out_shape=jax.ShapeDtypeStruct((E, 1), F32),
        compiler_params=pltpu.CompilerParams(
            dimension_semantics=("arbitrary",)),
    )(pp, pp)


def _tc_edge_mlp(hsd, d2, A, B, c1, b1, W2, b2, sW, sb, E, H, blk, d2_off=0):
    nb = E // blk

    def body(hs_r, hd_r, d2_r, A_r, B_r, c1_r, b1_r, W2_r, b2_r, sW_r, sb_r,
             o_r):
        z = jnp.dot(hs_r[...].astype(BF), A_r[...], preferred_element_type=F32)
        z = z + jnp.dot(hd_r[...].astype(BF), B_r[...],
                        preferred_element_type=F32)
        z = z + d2_r[...] * c1_r[...] + b1_r[...]
        m1 = _silu(z)
        z2 = jnp.dot(m1.astype(jnp.bfloat16), W2_r[...],
                     preferred_element_type=F32) + b2_r[...]
        m = _silu(z2)
        g = jax.nn.sigmoid(jnp.dot(m, sW_r[...],
                                   preferred_element_type=F32) + sb_r[...])
        o_r[...] = m * g

    return pl.pallas_call(
        body,
        grid=(nb,),
        in_specs=[
            pl.BlockSpec((blk, H), lambda i: (i, 0)),
            pl.BlockSpec((blk, H), lambda i, nb=nb: (i + nb, 0)),
            pl.BlockSpec((blk, 1), lambda i, o=d2_off: (i + o, 0)),
            pl.BlockSpec((H, H), lambda i: (0, 0)),
            pl.BlockSpec((H, H), lambda i: (0, 0)),
            pl.BlockSpec((1, H), lambda i: (0, 0)),
            pl.BlockSpec((1, H), lambda i: (0, 0)),
            pl.BlockSpec((H, H), lambda i: (0, 0)),
            pl.BlockSpec((1, H), lambda i: (0, 0)),
            pl.BlockSpec((H, 1), lambda i: (0, 0)),
            pl.BlockSpec((1, 1), lambda i: (0, 0)),
        ],
        out_specs=pl.BlockSpec((blk, H), lambda i: (i, 0)),
        out_shape=jax.ShapeDtypeStruct((E, H), F32),
        compiler_params=pltpu.CompilerParams(
            dimension_semantics=("arbitrary",)),
    )(hsd, hsd, d2, A, B, c1, b1, W2, b2, sW, sb)


def _tc_node_mlp(p0, p1, h, W1, b1, W2, b2, blk):
    N, H = h.shape

    def body(p0_r, p1_r, h_r, W1_r, b1_r, W2_r, b2_r, o_r):
        hi = p0_r[...] + p1_r[...]
        a = _silu(jnp.dot(hi, W1_r[...], preferred_element_type=F32)
                  + b1_r[...])
        o_r[...] = (jnp.dot(a, W2_r[...], preferred_element_type=F32)
                    + b2_r[...] + h_r[...])

    return pl.pallas_call(
        body,
        grid=(N // blk,),
        in_specs=[
            pl.BlockSpec((blk, H), lambda i: (i, 0)),
            pl.BlockSpec((blk, H), lambda i: (i, 0)),
            pl.BlockSpec((blk, H), lambda i: (i, 0)),
            pl.BlockSpec((H, H), lambda i: (0, 0)),
            pl.BlockSpec((1, H), lambda i: (0, 0)),
            pl.BlockSpec((H, H), lambda i: (0, 0)),
            pl.BlockSpec((1, H), lambda i: (0, 0)),
        ],
        out_specs=pl.BlockSpec((blk, H), lambda i: (i, 0)),
        out_shape=jax.ShapeDtypeStruct((N, H), F32),
        compiler_params=pltpu.CompilerParams(
            dimension_semantics=("arbitrary",)),
    )(p0, p1, h, W1, b1.reshape(1, H), W2, b2.reshape(1, H))


def _tc_readout(h, oW1, ob1, oW2, ob2, rW1, rb1, rW2, rb2, blk):
    N, H = h.shape
    T = rW2.shape[1]
    nb = N // blk

    def body(h_r, oW1_r, ob1_r, oW2_r, ob2_r, rW1_r, rb1_r, rW2_r, rb2_r,
             o_r, sum_acc, max_acc):
        i = pl.program_id(0)
        a = _silu(jnp.dot(h_r[...], oW1_r[...], preferred_element_type=F32)
                  + ob1_r[...])
        ho = jnp.dot(a, oW2_r[...], preferred_element_type=F32) + ob2_r[...]
        ps = jnp.sum(ho, axis=0, keepdims=True)
        pm = jnp.max(ho, axis=0, keepdims=True)

        @pl.when(i == 0)
        def _():
            sum_acc[...] = ps
            max_acc[...] = pm

        @pl.when(i > 0)
        def _():
            sum_acc[...] = sum_acc[...] + ps
            max_acc[...] = jnp.maximum(max_acc[...], pm)

        @pl.when(i == nb - 1)
        def _():
            s = sum_acc[...]
            mx = max_acc[...]
            r = jnp.concatenate([s, s * (1.0 / N), mx], axis=1)
            t = jnp.dot(r, rW1_r[...], preferred_element_type=F32) + rb1_r[...]
            t = jnp.maximum(t, 0.0)
            o_r[...] = (jnp.dot(t, rW2_r[...], preferred_element_type=F32)
                        + rb2_r[...])

    return pl.pallas_call(
        body,
        grid=(nb,),
        in_specs=[
            pl.BlockSpec((blk, H), lambda i: (i, 0)),
            pl.BlockSpec((H, H), lambda i: (0, 0)),
            pl.BlockSpec((1, H), lambda i: (0, 0)),
            pl.BlockSpec((H, H), lambda i: (0, 0)),
            pl.BlockSpec((1, H), lambda i: (0, 0)),
            pl.BlockSpec((3 * H, H), lambda i: (0, 0)),
            pl.BlockSpec((1, H), lambda i: (0, 0)),
            pl.BlockSpec((H, T), lambda i: (0, 0)),
            pl.BlockSpec((1, T), lambda i: (0, 0)),
        ],
        out_specs=pl.BlockSpec((1, T), lambda i: (0, 0)),
        out_shape=jax.ShapeDtypeStruct((1, T), F32),
        scratch_shapes=[pltpu.VMEM((1, H), F32), pltpu.VMEM((1, H), F32)],
        compiler_params=pltpu.CompilerParams(
            dimension_semantics=("arbitrary",)),
    )(h, oW1, ob1.reshape(1, H), oW2, ob2.reshape(1, H),
      rW1, rb1.reshape(1, H), rW2, rb2.reshape(1, T))


BF = jnp.bfloat16
EBLK = 2000
NBLK = 2000


def kernel(x, pos, edge_index, params):
    N, D = x.shape
    E = edge_index.shape[1]
    H = params["in_W"].shape[1]

    h = _tc_input_mlp(x, params["in_W"], params["in_b"], NBLK)

    idx_flat = edge_index.reshape(2 * E)
    dst = edge_index[1]
    pos_pad = jnp.concatenate(
        [pos, jnp.zeros((N, 128 - pos.shape[1]), F32)], axis=1)

    pp = _sc_gather(pos_pad, idx_flat, GCHUNK)
    d2 = _tc_d2(pp, E, EBLK)

    zeros = jnp.zeros((N, H), F32)
    EA = E * 3 // 5
    assert EA % (NW * GCHUNK) == 0 and (E - EA) % (NW * GCHUNK) == 0
    idxA = jnp.concatenate([edge_index[0, :EA], edge_index[1, :EA]])
    idxB = jnp.concatenate([edge_index[0, EA:], edge_index[1, EA:]])
    dstA = edge_index[1, :EA]
    dstB = edge_index[1, EA:]
    for lp in params["layers"]:
        eargs = (lp["mW1"][:H].astype(BF), lp["mW1"][H:2 * H].astype(BF),
                 lp["mW1"][2 * H:2 * H + 1],
                 lp["mb1"].reshape(1, H), lp["mW2"].astype(BF),
                 lp["mb2"].reshape(1, H),
                 lp["sW"], lp["sb"].reshape(1, 1))
        hsdA = _sc_gather(h, idxA, GCHUNK)
        hsdB = _sc_gather(h, idxB, GCHUNK)
        mA = _tc_edge_mlp(hsdA, d2, *eargs, EA, H, EBLK)
        mB = _tc_edge_mlp(hsdB, d2, *eargs, E - EA, H, EBLK,
                          d2_off=EA // EBLK)
        q0, q1 = _sc_scatter_add(mA, dstA, h, zeros, SCHUNK)
        p0, p1 = _sc_scatter_add(mB, dstB, q0, q1, SCHUNK)
        h = _tc_node_mlp(p0, p1, h, lp["uW1"], lp["ub1"], lp["uW2"],
                         lp["ub2"], NBLK)

    return _tc_readout(h, params["oW1"], params["ob1"], params["oW2"],
                       params["ob2"], params["rW1"], params["rb1"],
                       params["rW2"], params["rb2"], NBLK)
